# baseline (speedup 1.0000x reference)
import functools

import jax
import jax.numpy as jnp
from jax import lax
from jax.experimental import pallas as pl
from jax.experimental.pallas import tpu as pltpu
from jax.experimental.pallas import tpu_sc as plsc

NWORDS = 1000000
NTAGS = 64
NTOK = 16384

NC = 2
NS = 16
LANES = 16
B_PER_SC = NTOK // NC
B_PER_W = B_PER_SC // NS
SCHUNK = 128
NSCHUNK = B_PER_W // SCHUNK

W_PAD = 1 << 20
W_HALF = W_PAD // 2
WH_PER_TILE = W_HALF // NS
ZBUF = 8192

BW = 32768
_GRID = pl.cdiv(NWORDS, BW)
NB_A = W_HALF // BW
NB_B = _GRID - NB_A

_mesh = plsc.VectorSubcoreMesh(
    core_axis_name="c", subcore_axis_name="s", num_cores=NC, num_subcores=NS
)


def _make_hist(lo):
    def body(x_hbm, z_hbm, o1_hbm, out_hbm, idx_v, ones_v, zbuf_v, acc_sh):
        cid = lax.axis_index("c")
        sid = lax.axis_index("s")

        pltpu.sync_copy(z_hbm, zbuf_v)
        for k in range(WH_PER_TILE // ZBUF):
            pltpu.sync_copy(
                zbuf_v, acc_sh.at[pl.ds(sid * WH_PER_TILE + k * ZBUF, ZBUF)]
            )

        pltpu.sync_copy(o1_hbm, ones_v)

        base = cid * B_PER_SC + sid * B_PER_W
        for j in range(NSCHUNK):
            pltpu.sync_copy(
                x_hbm.at[pl.ds(base + j * SCHUNK, SCHUNK)], idx_v.at[j]
            )
        for j in range(NSCHUNK):
            for g in range(SCHUNK // LANES):
                sl = pl.ds(g * LANES, LANES)
                v = idx_v[j, sl] - lo
                ok = (v >= 0) & (v < W_HALF)
                idx_v[j, sl] = jnp.where(ok, v, W_HALF)
        plsc.subcore_barrier()
        for j in range(NSCHUNK):
            pltpu.sync_copy(ones_v, acc_sh.at[idx_v.at[j]], add=True)
        plsc.subcore_barrier()
        pltpu.sync_copy(
            acc_sh.at[pl.ds(sid * WH_PER_TILE, WH_PER_TILE)],
            out_hbm.at[pl.ds(cid * W_HALF + sid * WH_PER_TILE, WH_PER_TILE)],
        )

    return functools.partial(
        pl.kernel,
        mesh=_mesh,
        out_type=jax.ShapeDtypeStruct((NC * W_HALF,), jnp.float32),
        scratch_types=[
            pltpu.VMEM((NSCHUNK, SCHUNK), jnp.int32),
            pltpu.VMEM((SCHUNK,), jnp.float32),
            pltpu.VMEM((ZBUF,), jnp.float32),
            pltpu.VMEM_SHARED((W_HALF + 2 * LANES,), jnp.float32),
        ],
    )(body)


_hist_lo = _make_hist(0)
_hist_hi = _make_hist(W_HALF)


def _mv_a_body(t_ref, c0_ref, c1_ref, o_ref, acc_ref):
    i = pl.program_id(0)

    @pl.when(i == 0)
    def _init():
        acc_ref[...] = jnp.zeros_like(acc_ref)

    acc_ref[...] += t_ref[...] * (c0_ref[...] + c1_ref[...])

    @pl.when(i == NB_A - 1)
    def _fin():
        o_ref[...] = jnp.sum(acc_ref[...], axis=1)[None, :]


def _mv_b_body(t_ref, c0_ref, c1_ref, p_ref, b_ref, o_ref, acc_ref):
    i = pl.program_id(0)

    @pl.when(i == 0)
    def _init():
        acc_ref[...] = jnp.zeros_like(acc_ref)

    acc_ref[...] += t_ref[...] * (c0_ref[...] + c1_ref[...])

    @pl.when(i == NB_B - 1)
    def _fin():
        o_ref[...] = (
            jnp.sum(acc_ref[...], axis=1)[None, :] + p_ref[...] + b_ref[...]
        )


def kernel(x, table, bias):
    xi = x.astype(jnp.int32)
    zeros = jnp.zeros((ZBUF,), jnp.float32)
    ones = jnp.ones((SCHUNK,), jnp.float32)
    c_lo = _hist_lo(xi, zeros, ones).reshape(1, NC * W_HALF)
    c_hi = _hist_hi(xi, zeros, ones).reshape(1, NC * W_HALF)
    table_t = table.T
    part = pl.pallas_call(
        _mv_a_body,
        grid=(NB_A,),
        in_specs=[
            pl.BlockSpec((NTAGS, BW), lambda i: (0, i)),
            pl.BlockSpec((1, BW), lambda i: (0, i)),
            pl.BlockSpec((1, BW), lambda i: (0, W_HALF // BW + i)),
        ],
        out_specs=pl.BlockSpec((1, NTAGS), lambda i: (0, 0)),
        out_shape=jax.ShapeDtypeStruct((1, NTAGS), jnp.float32),
        scratch_shapes=[pltpu.VMEM((NTAGS, BW), jnp.float32)],
        compiler_params=pltpu.CompilerParams(
            dimension_semantics=("arbitrary",)
        ),
    )(table_t, c_lo, c_lo)
    return pl.pallas_call(
        _mv_b_body,
        grid=(NB_B,),
        in_specs=[
            pl.BlockSpec((NTAGS, BW), lambda i: (0, NB_A + i)),
            pl.BlockSpec((1, BW), lambda i: (0, i)),
            pl.BlockSpec((1, BW), lambda i: (0, W_HALF // BW + i)),
            pl.BlockSpec((1, NTAGS), lambda i: (0, 0)),
            pl.BlockSpec((1, NTAGS), lambda i: (0, 0)),
        ],
        out_specs=pl.BlockSpec((1, NTAGS), lambda i: (0, 0)),
        out_shape=jax.ShapeDtypeStruct((1, NTAGS), jnp.float32),
        scratch_shapes=[pltpu.VMEM((NTAGS, BW), jnp.float32)],
        compiler_params=pltpu.CompilerParams(
            dimension_semantics=("arbitrary",)
        ),
    )(table_t, c_hi, c_hi, part, bias.reshape(1, NTAGS))


# speedup vs baseline: 1.0675x; 1.0675x over previous
import functools

import jax
import jax.numpy as jnp
from jax import lax
from jax.experimental import pallas as pl
from jax.experimental.pallas import tpu as pltpu
from jax.experimental.pallas import tpu_sc as plsc

NWORDS = 1000000
NTAGS = 64
NTOK = 16384

NC = 2
NS = 16
LANES = 16
B_PER_SC = NTOK // NC
B_PER_W = B_PER_SC // NS
SCHUNK = 128
NSCHUNK = B_PER_W // SCHUNK

W_PAD = 1 << 20
W_PER_TILE = W_PAD // NS
ZBUF = 8192

_mesh = plsc.VectorSubcoreMesh(
    core_axis_name="c", subcore_axis_name="s", num_cores=NC, num_subcores=NS
)


def _hist_body(x_hbm, z_hbm, o1_hbm, out_hbm, idx_v, ones_v, zbuf_v, acc_sh):
    cid = lax.axis_index("c")
    sid = lax.axis_index("s")

    pltpu.sync_copy(z_hbm, zbuf_v)
    for k in range(W_PER_TILE // ZBUF):
        pltpu.sync_copy(
            zbuf_v, acc_sh.at[pl.ds(sid * W_PER_TILE + k * ZBUF, ZBUF)]
        )

    pltpu.sync_copy(o1_hbm, ones_v)

    base = cid * B_PER_SC + sid * B_PER_W
    for j in range(NSCHUNK):
        pltpu.sync_copy(x_hbm.at[pl.ds(base + j * SCHUNK, SCHUNK)], idx_v.at[j])
    plsc.subcore_barrier()
    for j in range(NSCHUNK):
        pltpu.sync_copy(ones_v, acc_sh.at[idx_v.at[j]], add=True)
    plsc.subcore_barrier()
    pltpu.sync_copy(
        acc_sh.at[pl.ds(sid * W_PER_TILE, W_PER_TILE)],
        out_hbm.at[pl.ds(cid * W_PAD + sid * W_PER_TILE, W_PER_TILE)],
    )


_hist = functools.partial(
    pl.kernel,
    mesh=_mesh,
    out_type=jax.ShapeDtypeStruct((NC * W_PAD,), jnp.float32),
    scratch_types=[
        pltpu.VMEM((NSCHUNK, SCHUNK), jnp.int32),
        pltpu.VMEM((SCHUNK,), jnp.float32),
        pltpu.VMEM((ZBUF,), jnp.float32),
        pltpu.VMEM_SHARED((W_PAD,), jnp.float32),
    ],
)(_hist_body)

BW = 32768
_GRID = pl.cdiv(NWORDS, BW)


def _matvec_body(t_ref, c0_ref, c1_ref, b_ref, o_ref, acc_ref):
    i = pl.program_id(0)

    @pl.when(i == 0)
    def _init():
        acc_ref[...] = jnp.zeros_like(acc_ref)

    c = (c0_ref[...] + c1_ref[...]).astype(jnp.float32)
    acc_ref[...] += t_ref[...] * c

    @pl.when(i == _GRID - 1)
    def _fin():
        o_ref[...] = jnp.sum(acc_ref[...], axis=1)[None, :] + b_ref[...]


def kernel(x, table, bias):
    counts = _hist(
        x.astype(jnp.int32),
        jnp.zeros((ZBUF,), jnp.float32),
        jnp.ones((SCHUNK,), jnp.float32),
    )
    counts2 = counts.reshape(1, NC * W_PAD)
    table_t = table.T
    return pl.pallas_call(
        _matvec_body,
        grid=(_GRID,),
        in_specs=[
            pl.BlockSpec((NTAGS, BW), lambda i: (0, i)),
            pl.BlockSpec((1, BW), lambda i: (0, i)),
            pl.BlockSpec((1, BW), lambda i: (0, W_PAD // BW + i)),
            pl.BlockSpec((1, NTAGS), lambda i: (0, 0)),
        ],
        out_specs=pl.BlockSpec((1, NTAGS), lambda i: (0, 0)),
        out_shape=jax.ShapeDtypeStruct((1, NTAGS), jnp.float32),
        scratch_shapes=[pltpu.VMEM((NTAGS, BW), jnp.float32)],
        compiler_params=pltpu.CompilerParams(
            dimension_semantics=("arbitrary",)
        ),
    )(table_t, counts2, counts2, bias.reshape(1, NTAGS))
